# speedup vs baseline: 1.1983x; 1.1983x over previous
import functools

import jax
import jax.numpy as jnp
from jax import lax
from jax.experimental import pallas as pl
from jax.experimental.pallas import tpu as pltpu
from jax.experimental.pallas import tpu_sc as plsc

_B = 64
_S = 512
_DIM = 768
_NTOK = _B * _S

_NC = 2
_NS = 16
_NW = _NC * _NS
_TOK_PER_W = _NTOK // _NW
_CHUNK = 64
_NCHUNK = _TOK_PER_W // _CHUNK


def _sc_gather(table, ids):
  mesh = plsc.VectorSubcoreMesh(core_axis_name="c", subcore_axis_name="s")

  @functools.partial(
      pl.kernel,
      out_type=jax.ShapeDtypeStruct((_NTOK, _DIM), jnp.float32),
      mesh=mesh,
      scratch_types=[
          pltpu.VMEM((_TOK_PER_W,), jnp.int32),
          pltpu.VMEM((2, _CHUNK, _DIM), jnp.float32),
          pltpu.SemaphoreType.DMA,
          pltpu.SemaphoreType.DMA,
          pltpu.SemaphoreType.DMA,
          pltpu.SemaphoreType.DMA,
      ],
  )
  def k(table_hbm, idx_hbm, out_hbm, idx_v, rows_v, g0, g1, o0, o1):
    wid = lax.axis_index("s") * _NC + lax.axis_index("c")
    base = wid * _TOK_PER_W
    pltpu.sync_copy(idx_hbm.at[pl.ds(base, _TOK_PER_W)], idx_v)

    gsem = [g0, g1]
    osem = [o0, o1]

    def gather(i):
      return pltpu.async_copy(
          table_hbm.at[idx_v.at[pl.ds(i * _CHUNK, _CHUNK)]],
          rows_v.at[i % 2],
          gsem[i % 2],
      )

    pend_g = [None, None]
    pend_o = [None, None]
    pend_g[0] = gather(0)
    for i in range(_NCHUNK):
      b = i % 2
      nb = (i + 1) % 2
      if i + 1 < _NCHUNK:
        if pend_o[nb] is not None:
          pend_o[nb].wait()
        pend_g[nb] = gather(i + 1)
      pend_g[b].wait()
      pend_o[b] = pltpu.async_copy(
          rows_v.at[b],
          out_hbm.at[pl.ds(base + i * _CHUNK, _CHUNK)],
          osem[b],
      )
    pend_o[0].wait()
    pend_o[1].wait()

  return k(table, ids)


def _tc_addln(words, pos, tt, gamma, beta):
  def body(w_ref, p_ref, t_ref, g_ref, b_ref, o_ref):
    x = w_ref[...] + p_ref[...] + t_ref[0][None, :]
    m = jnp.mean(x, axis=-1, keepdims=True)
    c = x - m
    v = jnp.mean(c * c, axis=-1, keepdims=True)
    o_ref[...] = c / jnp.sqrt(v + 1e-12) * g_ref[...] + b_ref[...]

  return pl.pallas_call(
      body,
      grid=(_B,),
      in_specs=[
          pl.BlockSpec((_S, _DIM), lambda i: (i, 0)),
          pl.BlockSpec((_S, _DIM), lambda i: (0, 0)),
          pl.BlockSpec((2, _DIM), lambda i: (0, 0)),
          pl.BlockSpec((1, _DIM), lambda i: (0, 0)),
          pl.BlockSpec((1, _DIM), lambda i: (0, 0)),
      ],
      out_specs=pl.BlockSpec((_S, _DIM), lambda i: (i, 0)),
      out_shape=jax.ShapeDtypeStruct((_NTOK, _DIM), jnp.float32),
  )(words, pos, tt, gamma, beta)


def kernel(input_ids, word_embeddings, position_embeddings,
           token_type_embeddings, ln_gamma, ln_beta):
  ids = input_ids.reshape(-1).astype(jnp.int32)
  words = _sc_gather(word_embeddings, ids)
  out = _tc_addln(
      words,
      position_embeddings,
      token_type_embeddings,
      ln_gamma.reshape(1, _DIM),
      ln_beta.reshape(1, _DIM),
  )
  return out.reshape(_B, _S, _DIM)
